# Initial kernel scaffold; baseline (speedup 1.0000x reference)
import jax
import jax.numpy as jnp
from jax.experimental import pallas as pl


def kernel(x1, x2, edge_index1, edge_index2, edge_attr1, edge_attr2, embed, edge_embed, W, b, gate_W, gate_b, fc1_W, fc1_b, fc2_W, fc2_b):
    raise NotImplementedError("write your pallas kernel here")


# speedup vs baseline: 11.4026x; 11.4026x over previous
import functools

import jax
import jax.numpy as jnp
from jax import lax
from jax.experimental import pallas as pl
from jax.experimental.pallas import tpu as pltpu
from jax.experimental.pallas import tpu_sc as plsc

N = 10000
NPAD = 10240
D = 256
E = 160000
KCH = 128
NCH = 84
ET = NCH * KCH
V = 100000
EV = 20
NT = 16
RPT = NPAD // NT
L = 16

f32 = jnp.float32
i32 = jnp.int32

_mesh = plsc.VectorSubcoreMesh(core_axis_name="c", subcore_axis_name="s")
_sc_params = pltpu.CompilerParams(needs_layout_passes=False)


def _bcast_lane(vec, lane):
    return lax.broadcast_in_dim(
        lax.squeeze(lax.slice_in_dim(vec, lane, lane + 1), (0,)), (L,), ())


@functools.partial(
    pl.kernel,
    out_type=jax.ShapeDtypeStruct((2, NPAD, D), f32),
    mesh=_mesh,
    compiler_params=_sc_params,
    scratch_types=[
        pltpu.VMEM((RPT,), i32),
        pltpu.VMEM((160, D), f32),
        pltpu.SemaphoreType.DMA,
    ],
)
def _embed_gather(embed_hbm, idx_hbm, out_hbm, idx_v, rows_v, sem):
    c = lax.axis_index("c")
    s = lax.axis_index("s")
    base = s * RPT
    pltpu.sync_copy(idx_hbm.at[c, pl.ds(base, RPT)], idx_v)

    def chunk(i, _):
        pltpu.async_copy(embed_hbm.at[idx_v.at[pl.ds(i * 160, 160)]], rows_v, sem).wait()
        pltpu.sync_copy(rows_v, out_hbm.at[c, pl.ds(base + i * 160, 160)])
        return ()

    lax.fori_loop(0, RPT // 160, chunk, (), unroll=False)


def _xw_body(x_ref, w_ref, ee_ref, h0_ref, h1_ref, rmax_ref):
    xb = x_ref[0]
    h0_ref[0] = jnp.dot(xb, w_ref[:, :128], preferred_element_type=f32)
    h1_ref[0] = jnp.dot(xb, w_ref[:, 128:], preferred_element_type=f32)
    rm = jnp.max(ee_ref[...], axis=1)
    rmax_ref[...] = jnp.concatenate(
        [rm, jnp.ones((1,), f32), jnp.zeros((127 - EV,), f32)]).reshape(1, 128)


def _dense_xw(x12, W, edge_embed):
    nblk = 8
    rb = NPAD // nblk
    h0, h1, rmax = pl.pallas_call(
        _xw_body,
        grid=(2, nblk),
        in_specs=[
            pl.BlockSpec((1, rb, D), lambda b, i: (b, i, 0)),
            pl.BlockSpec((D, D), lambda b, i: (0, 0)),
            pl.BlockSpec((EV, D), lambda b, i: (0, 0)),
        ],
        out_specs=[
            pl.BlockSpec((1, rb, 128), lambda b, i: (b, i, 0)),
            pl.BlockSpec((1, rb, 128), lambda b, i: (b, i, 0)),
            pl.BlockSpec((1, 128), lambda b, i: (0, 0)),
        ],
        out_shape=[
            jax.ShapeDtypeStruct((2, NPAD, 128), f32),
            jax.ShapeDtypeStruct((2, NPAD, 128), f32),
            jax.ShapeDtypeStruct((1, 128), f32),
        ],
    )(x12, W, edge_embed)
    return h0, h1, rmax


@functools.partial(
    pl.kernel,
    out_type=jax.ShapeDtypeStruct((2, NT, NCH, 1, KCH), f32),
    mesh=_mesh,
    compiler_params=_sc_params,
    scratch_types=[
        pltpu.VMEM((NCH, KCH), i32),
        pltpu.VMEM((NCH, KCH), i32),
        pltpu.VMEM((NCH, KCH), i32),
        pltpu.VMEM((NCH, 1, KCH), f32),
        pltpu.VMEM((NPAD,), f32),
        pltpu.VMEM((80, 128), f32),
        pltpu.VMEM((80, 128), f32),
        pltpu.VMEM((NPAD,), f32),
        pltpu.VMEM((1, 80), i32),
        pltpu.VMEM((128,), f32),
        pltpu.VMEM_SHARED((80, 128), f32),
    ],
)
def _deg_norm(srcs, dsts, eas, rmax_hbm, norm_hbm,
              srcb, dstb, eab, normb, hist, hist2, degb, disb, idx80,
              rmaxb, deg_sh):
    c = lax.axis_index("c")
    s = lax.axis_index("s")

    pltpu.sync_copy(srcs.at[c, s], srcb)
    pltpu.sync_copy(dsts.at[c, s], dstb)
    pltpu.sync_copy(eas.at[c, s], eab)
    pltpu.sync_copy(rmax_hbm, rmaxb)

    for q in range(5):
        idx80[0, pl.ds(q * L, L)] = lax.broadcasted_iota(i32, (L,), 0) + q * L

    def zrow(r, _):
        hist[pl.ds(r * L, L)] = jnp.zeros((L,), f32)
        return ()
    lax.fori_loop(0, NPAD // L, zrow, ())

    def z2row(r, _):
        for q in range(8):
            hist2[r, pl.ds(q * L, L)] = jnp.zeros((L,), f32)
        return ()
    lax.fori_loop(0, 80, z2row, ())

    @pl.when(s == 0)
    def _():
        pltpu.sync_copy(hist2, deg_sh)
    plsc.subcore_barrier()

    def hrow(r, _):
        for q in range(KCH // L):
            dv = dstb[r, pl.ds(q * L, L)]
            av = eab[r, pl.ds(q * L, L)]
            wv = plsc.load_gather(rmaxb, [av])
            plsc.addupdate_scatter(hist, [dv], wv)
        return ()
    lax.fori_loop(0, NCH, hrow, ())

    def crow(r, _):
        for q in range(8):
            hist2[r, pl.ds(q * L, L)] = hist[pl.ds(r * 128 + q * L, L)]
        return ()
    lax.fori_loop(0, 80, crow, ())
    pltpu.sync_copy(hist2, deg_sh.at[idx80.at[0]], add=True)
    plsc.subcore_barrier()

    pltpu.sync_copy(deg_sh, degb)

    def drow(r, _):
        for q in range(8):
            d = degb[r, pl.ds(q * L, L)]
            pos = d > 0.0
            dsafe = jnp.where(pos, d, jnp.ones((L,), f32))
            y = plsc.bitcast(jnp.int32(0x5F3759DF) - (plsc.bitcast(dsafe, i32) >> 1), f32)
            for _ in range(3):
                y = y * (1.5 - 0.5 * dsafe * y * y)
            dis = jnp.where(pos, y, jnp.zeros((L,), f32))
            disb[pl.ds(r * 128 + q * L, L)] = dis
        return ()
    lax.fori_loop(0, 80, drow, ())

    def nrow(r, _):
        for q in range(KCH // L):
            sv = srcb[r, pl.ds(q * L, L)]
            dv = dstb[r, pl.ds(q * L, L)]
            av = eab[r, pl.ds(q * L, L)]
            wv = plsc.load_gather(rmaxb, [av])
            ds_ = plsc.load_gather(disb, [sv])
            dd_ = plsc.load_gather(disb, [dv])
            normb[r, 0, pl.ds(q * L, L)] = ds_ * wv * dd_
        return ()
    lax.fori_loop(0, NCH, nrow, ())
    pltpu.sync_copy(normb, norm_hbm.at[c, s])


@functools.partial(
    pl.kernel,
    out_type=jax.ShapeDtypeStruct((2, 2, NPAD, 128), f32),
    mesh=_mesh,
    compiler_params=_sc_params,
    scratch_types=[
        pltpu.VMEM((NCH, KCH), i32),
        pltpu.VMEM((NCH, KCH), i32),
        pltpu.VMEM((KCH,), f32),
        pltpu.VMEM((KCH, 128), f32),
        pltpu.VMEM_SHARED((NPAD, 128), f32),
        pltpu.SemaphoreType.DMA,
        pltpu.SemaphoreType.DMA,
    ],
)
def _messages(xw0, xw1, srcs, dsts, norm_hbm, m_hbm,
              srcv, dstv, normc, rows_v, acc_sh, sem, sem2):
    c = lax.axis_index("c")
    s = lax.axis_index("s")

    pltpu.sync_copy(srcs.at[c, s], srcv)
    pltpu.sync_copy(dsts.at[c, s], dstv)

    for h, xwh in ((0, xw0), (1, xw1)):
        def zrow(r, _):
            for q in range(8):
                rows_v[r, pl.ds(q * L, L)] = jnp.zeros((L,), f32)
            return ()
        lax.fori_loop(0, KCH, zrow, ())
        for i in range(RPT // KCH):
            pltpu.sync_copy(rows_v, acc_sh.at[pl.ds(s * RPT + i * KCH, KCH)])
        rem = RPT - (RPT // KCH) * KCH
        if rem:
            pltpu.sync_copy(rows_v.at[pl.ds(0, rem)],
                            acc_sh.at[pl.ds(s * RPT + (RPT // KCH) * KCH, rem)])
        plsc.subcore_barrier()

        def chunk(j, _):
            ncp = pltpu.async_copy(norm_hbm.at[c, s, j, 0], normc, sem2)
            rcp = pltpu.async_copy(xwh.at[c].at[srcv.at[j]], rows_v, sem)
            ncp.wait()
            rcp.wait()
            for g in range(KCH // L):
                nv = normc[pl.ds(g * L, L)]
                for k16 in range(L):
                    k = g * L + k16
                    nsp = _bcast_lane(nv, k16)
                    for q in range(8):
                        rows_v[k, pl.ds(q * L, L)] = rows_v[k, pl.ds(q * L, L)] * nsp
            pltpu.sync_copy(rows_v, acc_sh.at[dstv.at[j]], add=True)
            return ()
        lax.fori_loop(0, NCH, chunk, (), unroll=False)
        plsc.subcore_barrier()

        pltpu.sync_copy(acc_sh.at[pl.ds(s * RPT, RPT)],
                        m_hbm.at[c, h, pl.ds(s * RPT, RPT)])
        if h == 0:
            plsc.subcore_barrier()


def _attn_body(m_ref, b2_ref, gw_ref, gb_ref, f1w_ref, f1b_ref, f2w_ref,
               f2b_ref, out_ref):
    iota = lax.broadcasted_iota(i32, (NPAD, 1), 0)
    mask = iota < N
    pooled = []
    for bidx in range(2):
        m0 = jnp.maximum(m_ref[bidx, 0] + b2_ref[0], 0.0)
        m1 = jnp.maximum(m_ref[bidx, 1] + b2_ref[1], 0.0)
        g = (jnp.dot(m0, gw_ref[:128, :], preferred_element_type=f32)
             + jnp.dot(m1, gw_ref[128:, :], preferred_element_type=f32)
             + gb_ref[0, 0])
        g = jax.nn.sigmoid(g)
        gmax = jnp.max(jnp.where(mask, g, -jnp.inf))
        e = jnp.where(mask, jnp.exp(g - gmax), 0.0)
        alpha = e / jnp.sum(e)
        p0 = jnp.sum(alpha * m0, axis=0, keepdims=True)
        p1 = jnp.sum(alpha * m1, axis=0, keepdims=True)
        pooled.append((p0, p1))
    (p10, p11), (p20, p21) = pooled
    h = (jnp.dot(p10, f1w_ref[:128, :], preferred_element_type=f32)
         + jnp.dot(p11, f1w_ref[128:256, :], preferred_element_type=f32)
         + jnp.dot(p20, f1w_ref[256:384, :], preferred_element_type=f32)
         + jnp.dot(p21, f1w_ref[384:, :], preferred_element_type=f32)
         + f1b_ref[...])
    h = jnp.maximum(h, 0.0)
    out_ref[...] = jnp.dot(h, f2w_ref[...], preferred_element_type=f32) + f2b_ref[0, 0]


def _attention_mlp(m, b, gate_W, gate_b, fc1_W, fc1_b, fc2_W, fc2_b):
    return pl.pallas_call(
        _attn_body,
        out_shape=jax.ShapeDtypeStruct((1, 1), f32),
    )(m, b.reshape(2, 128), gate_W, gate_b.reshape(1, 1), fc1_W,
      fc1_b.reshape(1, D), fc2_W, fc2_b.reshape(1, 1))


def _edge_arrays(ei, ea_col):
    self_idx = jnp.arange(NPAD, dtype=i32).reshape(NT, RPT)
    pad_i = jnp.full((NT, 112), N, i32)
    src = jnp.concatenate(
        [ei[0].astype(i32).reshape(NT, E // NT), self_idx, pad_i], axis=1)
    dst = jnp.concatenate(
        [ei[1].astype(i32).reshape(NT, E // NT), self_idx, pad_i], axis=1)
    ea = jnp.concatenate(
        [ea_col.astype(i32).reshape(NT, E // NT),
         jnp.full((NT, RPT), EV, i32), jnp.full((NT, 112), EV, i32)], axis=1)
    return (src.reshape(NT, NCH, KCH), dst.reshape(NT, NCH, KCH),
            ea.reshape(NT, NCH, KCH))


def kernel(x1, x2, edge_index1, edge_index2, edge_attr1, edge_attr2,
           embed, edge_embed, W, b, gate_W, gate_b, fc1_W, fc1_b, fc2_W, fc2_b):
    pad0 = jnp.zeros((NPAD - N,), i32)
    idx = jnp.stack([jnp.concatenate([x1[:, 0].astype(i32), pad0]),
                     jnp.concatenate([x2[:, 0].astype(i32), pad0])])
    s1, d1, a1 = _edge_arrays(edge_index1, edge_attr1[:, 0])
    s2, d2, a2 = _edge_arrays(edge_index2, edge_attr2[:, 0])
    srcs = jnp.stack([s1, s2])
    dsts = jnp.stack([d1, d2])
    eas = jnp.stack([a1, a2])

    x12 = _embed_gather(embed, idx)
    xw0, xw1, rmax = _dense_xw(x12, W, edge_embed)
    norm = _deg_norm(srcs, dsts, eas, rmax.reshape(128))
    m = _messages(xw0, xw1, srcs, dsts, norm)
    out = _attention_mlp(m, b, gate_W, gate_b, fc1_W, fc1_b, fc2_W, fc2_b)
    return out.reshape(-1)


# speedup vs baseline: 13.1874x; 1.1565x over previous
import functools

import jax
import jax.numpy as jnp
from jax import lax
from jax.experimental import pallas as pl
from jax.experimental.pallas import tpu as pltpu
from jax.experimental.pallas import tpu_sc as plsc

N = 10000
NPAD = 10240
D = 256
E = 160000
KCH = 128
NCH = 84
ET = NCH * KCH
V = 100000
EV = 20
NT = 16
RPT = NPAD // NT
L = 16

f32 = jnp.float32
i32 = jnp.int32

_mesh = plsc.VectorSubcoreMesh(core_axis_name="c", subcore_axis_name="s")
_sc_params = pltpu.CompilerParams(needs_layout_passes=False)


def _bcast_lane(vec, lane):
    return lax.broadcast_in_dim(
        lax.squeeze(lax.slice_in_dim(vec, lane, lane + 1), (0,)), (L,), ())


@functools.partial(
    pl.kernel,
    out_type=jax.ShapeDtypeStruct((2, NPAD, D), f32),
    mesh=_mesh,
    compiler_params=_sc_params,
    scratch_types=[
        pltpu.VMEM((RPT,), i32),
        pltpu.VMEM((160, D), f32),
        pltpu.SemaphoreType.DMA,
    ],
)
def _embed_gather(embed_hbm, idx_hbm, out_hbm, idx_v, rows_v, sem):
    c = lax.axis_index("c")
    s = lax.axis_index("s")
    base = s * RPT
    pltpu.sync_copy(idx_hbm.at[c, pl.ds(base, RPT)], idx_v)

    def chunk(i, _):
        pltpu.async_copy(embed_hbm.at[idx_v.at[pl.ds(i * 160, 160)]], rows_v, sem).wait()
        pltpu.sync_copy(rows_v, out_hbm.at[c, pl.ds(base + i * 160, 160)])
        return ()

    lax.fori_loop(0, RPT // 160, chunk, (), unroll=False)


def _xw_body(x_ref, w_ref, ee_ref, h0_ref, h1_ref, rmax_ref):
    xb = x_ref[0]
    h0_ref[0] = jnp.dot(xb, w_ref[:, :128], preferred_element_type=f32)
    h1_ref[0] = jnp.dot(xb, w_ref[:, 128:], preferred_element_type=f32)
    rm = jnp.max(ee_ref[...], axis=1)
    rmax_ref[...] = jnp.concatenate(
        [rm, jnp.ones((1,), f32), jnp.zeros((127 - EV,), f32)]).reshape(1, 128)


def _dense_xw(x12, W, edge_embed):
    nblk = 8
    rb = NPAD // nblk
    h0, h1, rmax = pl.pallas_call(
        _xw_body,
        grid=(2, nblk),
        in_specs=[
            pl.BlockSpec((1, rb, D), lambda b, i: (b, i, 0)),
            pl.BlockSpec((D, D), lambda b, i: (0, 0)),
            pl.BlockSpec((EV, D), lambda b, i: (0, 0)),
        ],
        out_specs=[
            pl.BlockSpec((1, rb, 128), lambda b, i: (b, i, 0)),
            pl.BlockSpec((1, rb, 128), lambda b, i: (b, i, 0)),
            pl.BlockSpec((1, 128), lambda b, i: (0, 0)),
        ],
        out_shape=[
            jax.ShapeDtypeStruct((2, NPAD, 128), f32),
            jax.ShapeDtypeStruct((2, NPAD, 128), f32),
            jax.ShapeDtypeStruct((1, 128), f32),
        ],
    )(x12, W, edge_embed)
    return h0, h1, rmax


@functools.partial(
    pl.kernel,
    out_type=jax.ShapeDtypeStruct((2, NT, NCH, 1, KCH), f32),
    mesh=_mesh,
    compiler_params=_sc_params,
    scratch_types=[
        pltpu.VMEM((NCH, KCH), i32),
        pltpu.VMEM((NCH, KCH), i32),
        pltpu.VMEM((NCH, KCH), i32),
        pltpu.VMEM((NCH, 1, KCH), f32),
        pltpu.VMEM((NPAD,), f32),
        pltpu.VMEM((80, 128), f32),
        pltpu.VMEM((80, 128), f32),
        pltpu.VMEM((NPAD,), f32),
        pltpu.VMEM((1, 80), i32),
        pltpu.VMEM((128,), f32),
        pltpu.VMEM_SHARED((80, 128), f32),
    ],
)
def _deg_norm(srcs, dsts, eas, rmax_hbm, norm_hbm,
              srcb, dstb, eab, normb, hist, hist2, degb, disb, idx80,
              rmaxb, deg_sh):
    c = lax.axis_index("c")
    s = lax.axis_index("s")

    pltpu.sync_copy(srcs.at[c, s], srcb)
    pltpu.sync_copy(dsts.at[c, s], dstb)
    pltpu.sync_copy(eas.at[c, s], eab)
    pltpu.sync_copy(rmax_hbm, rmaxb)

    for q in range(5):
        idx80[0, pl.ds(q * L, L)] = lax.broadcasted_iota(i32, (L,), 0) + q * L

    def zrow(r, _):
        hist[pl.ds(r * L, L)] = jnp.zeros((L,), f32)
        return ()
    lax.fori_loop(0, NPAD // L, zrow, ())

    def z2row(r, _):
        for q in range(8):
            hist2[r, pl.ds(q * L, L)] = jnp.zeros((L,), f32)
        return ()
    lax.fori_loop(0, 80, z2row, ())

    @pl.when(s == 0)
    def _():
        pltpu.sync_copy(hist2, deg_sh)
    plsc.subcore_barrier()

    def hrow(r, _):
        for q in range(KCH // L):
            dv = dstb[r, pl.ds(q * L, L)]
            av = eab[r, pl.ds(q * L, L)]
            wv = plsc.load_gather(rmaxb, [av])
            plsc.addupdate_scatter(hist, [dv], wv)
        return ()
    lax.fori_loop(0, NCH, hrow, ())

    def crow(r, _):
        for q in range(8):
            hist2[r, pl.ds(q * L, L)] = hist[pl.ds(r * 128 + q * L, L)]
        return ()
    lax.fori_loop(0, 80, crow, ())
    pltpu.sync_copy(hist2, deg_sh.at[idx80.at[0]], add=True)
    plsc.subcore_barrier()

    pltpu.sync_copy(deg_sh, degb)

    def drow(r, _):
        for q in range(8):
            d = degb[r, pl.ds(q * L, L)]
            pos = d > 0.0
            dsafe = jnp.where(pos, d, jnp.ones((L,), f32))
            y = plsc.bitcast(jnp.int32(0x5F3759DF) - (plsc.bitcast(dsafe, i32) >> 1), f32)
            for _ in range(3):
                y = y * (1.5 - 0.5 * dsafe * y * y)
            dis = jnp.where(pos, y, jnp.zeros((L,), f32))
            disb[pl.ds(r * 128 + q * L, L)] = dis
        return ()
    lax.fori_loop(0, 80, drow, ())

    def nrow(r, _):
        for q in range(KCH // L):
            sv = srcb[r, pl.ds(q * L, L)]
            dv = dstb[r, pl.ds(q * L, L)]
            av = eab[r, pl.ds(q * L, L)]
            wv = plsc.load_gather(rmaxb, [av])
            ds_ = plsc.load_gather(disb, [sv])
            dd_ = plsc.load_gather(disb, [dv])
            normb[r, 0, pl.ds(q * L, L)] = ds_ * wv * dd_
        return ()
    lax.fori_loop(0, NCH, nrow, ())
    pltpu.sync_copy(normb, norm_hbm.at[c, s])


@functools.partial(
    pl.kernel,
    out_type=jax.ShapeDtypeStruct((2, 2, NPAD, 128), f32),
    mesh=_mesh,
    compiler_params=_sc_params,
    scratch_types=[
        pltpu.VMEM((NCH, KCH), i32),
        pltpu.VMEM((1, KCH), i32),
        pltpu.VMEM((1, KCH), i32),
        pltpu.VMEM((KCH,), f32),
        pltpu.VMEM((KCH,), f32),
        pltpu.VMEM((KCH, 128), f32),
        pltpu.VMEM((KCH, 128), f32),
        pltpu.VMEM_SHARED((NPAD, 128), f32),
        pltpu.SemaphoreType.DMA,
        pltpu.SemaphoreType.DMA,
    ],
)
def _messages(xw0, xw1, srcs, dsts, norm_hbm, m_hbm,
              srcv, dstA, dstB, nrmA, nrmB, rowsA, rowsB, acc_sh, semA, semB):
    c = lax.axis_index("c")
    s = lax.axis_index("s")

    pltpu.sync_copy(srcs.at[c, s], srcv)
    bufs = ((rowsA, nrmA, dstA, semA), (rowsB, nrmB, dstB, semB))

    for h, xwh in ((0, xw0), (1, xw1)):
        def _issue(j, buf):
            rows, nrm, dstb, sem = bufs[buf]
            pltpu.async_copy(norm_hbm.at[c, s, j, 0], nrm, sem)
            pltpu.async_copy(dsts.at[c, s, j], dstb, sem)
            pltpu.async_copy(xwh.at[c].at[srcv.at[j]], rows, sem)

        def _wait(j, buf):
            rows, nrm, dstb, sem = bufs[buf]
            pltpu.make_async_copy(norm_hbm.at[c, s, j, 0], nrm, sem).wait()
            pltpu.make_async_copy(dsts.at[c, s, j], dstb, sem).wait()
            pltpu.make_async_copy(xwh.at[c].at[srcv.at[j]], rows, sem).wait()

        def zrow(r, _):
            for q in range(8):
                rowsA[r, pl.ds(q * L, L)] = jnp.zeros((L,), f32)
            return ()
        lax.fori_loop(0, KCH, zrow, ())
        for i in range(RPT // KCH):
            pltpu.sync_copy(rowsA, acc_sh.at[pl.ds(s * RPT + i * KCH, KCH)])
        plsc.subcore_barrier()

        _issue(0, 0)

        def pair(jj, _):
            for b in range(2):
                j = 2 * jj + b
                rows, nrm, dstb, sem = bufs[b]

                @pl.when(j + 1 < NCH)
                def _():
                    _issue(j + 1, 1 - b)
                _wait(j, b)
                for g in range(KCH // L):
                    nv = nrm[pl.ds(g * L, L)]
                    for k16 in range(L):
                        k = g * L + k16
                        nsp = _bcast_lane(nv, k16)
                        for q in range(8):
                            rows[k, pl.ds(q * L, L)] = rows[k, pl.ds(q * L, L)] * nsp
                pltpu.sync_copy(rows, acc_sh.at[dstb.at[0]], add=True)
            return ()
        lax.fori_loop(0, NCH // 2, pair, (), unroll=False)
        plsc.subcore_barrier()

        pltpu.sync_copy(acc_sh.at[pl.ds(s * RPT, RPT)],
                        m_hbm.at[c, h, pl.ds(s * RPT, RPT)])
        if h == 0:
            plsc.subcore_barrier()


def _attn_body(m_ref, b2_ref, gw_ref, gb_ref, f1w_ref, f1b_ref, f2w_ref,
               f2b_ref, out_ref):
    iota = lax.broadcasted_iota(i32, (NPAD, 1), 0)
    mask = iota < N
    pooled = []
    for bidx in range(2):
        m0 = jnp.maximum(m_ref[bidx, 0] + b2_ref[0], 0.0)
        m1 = jnp.maximum(m_ref[bidx, 1] + b2_ref[1], 0.0)
        g = (jnp.dot(m0, gw_ref[:128, :], preferred_element_type=f32)
             + jnp.dot(m1, gw_ref[128:, :], preferred_element_type=f32)
             + gb_ref[0, 0])
        g = jax.nn.sigmoid(g)
        gmax = jnp.max(jnp.where(mask, g, -jnp.inf))
        e = jnp.where(mask, jnp.exp(g - gmax), 0.0)
        alpha = e / jnp.sum(e)
        p0 = jnp.sum(alpha * m0, axis=0, keepdims=True)
        p1 = jnp.sum(alpha * m1, axis=0, keepdims=True)
        pooled.append((p0, p1))
    (p10, p11), (p20, p21) = pooled
    h = (jnp.dot(p10, f1w_ref[:128, :], preferred_element_type=f32)
         + jnp.dot(p11, f1w_ref[128:256, :], preferred_element_type=f32)
         + jnp.dot(p20, f1w_ref[256:384, :], preferred_element_type=f32)
         + jnp.dot(p21, f1w_ref[384:, :], preferred_element_type=f32)
         + f1b_ref[...])
    h = jnp.maximum(h, 0.0)
    out_ref[...] = jnp.dot(h, f2w_ref[...], preferred_element_type=f32) + f2b_ref[0, 0]


def _attention_mlp(m, b, gate_W, gate_b, fc1_W, fc1_b, fc2_W, fc2_b):
    return pl.pallas_call(
        _attn_body,
        out_shape=jax.ShapeDtypeStruct((1, 1), f32),
    )(m, b.reshape(2, 128), gate_W, gate_b.reshape(1, 1), fc1_W,
      fc1_b.reshape(1, D), fc2_W, fc2_b.reshape(1, 1))


def _edge_arrays(ei, ea_col):
    self_idx = jnp.arange(NPAD, dtype=i32).reshape(NT, RPT)
    pad_i = jnp.full((NT, 112), N, i32)
    src = jnp.concatenate(
        [ei[0].astype(i32).reshape(NT, E // NT), self_idx, pad_i], axis=1)
    dst = jnp.concatenate(
        [ei[1].astype(i32).reshape(NT, E // NT), self_idx, pad_i], axis=1)
    ea = jnp.concatenate(
        [ea_col.astype(i32).reshape(NT, E // NT),
         jnp.full((NT, RPT), EV, i32), jnp.full((NT, 112), EV, i32)], axis=1)
    return (src.reshape(NT, NCH, KCH), dst.reshape(NT, NCH, KCH),
            ea.reshape(NT, NCH, KCH))


def kernel(x1, x2, edge_index1, edge_index2, edge_attr1, edge_attr2,
           embed, edge_embed, W, b, gate_W, gate_b, fc1_W, fc1_b, fc2_W, fc2_b):
    pad0 = jnp.zeros((NPAD - N,), i32)
    idx = jnp.stack([jnp.concatenate([x1[:, 0].astype(i32), pad0]),
                     jnp.concatenate([x2[:, 0].astype(i32), pad0])])
    s1, d1, a1 = _edge_arrays(edge_index1, edge_attr1[:, 0])
    s2, d2, a2 = _edge_arrays(edge_index2, edge_attr2[:, 0])
    srcs = jnp.stack([s1, s2])
    dsts = jnp.stack([d1, d2])
    eas = jnp.stack([a1, a2])

    x12 = _embed_gather(embed, idx)
    xw0, xw1, rmax = _dense_xw(x12, W, edge_embed)
    norm = _deg_norm(srcs, dsts, eas, rmax.reshape(128))
    m = _messages(xw0, xw1, srcs, dsts.reshape(2, NT, NCH, 1, KCH), norm)
    out = _attention_mlp(m, b, gate_W, gate_b, fc1_W, fc1_b, fc2_W, fc2_b)
    return out.reshape(-1)


# speedup vs baseline: 15.4943x; 1.1749x over previous
import functools

import jax
import jax.numpy as jnp
from jax import lax
from jax.experimental import pallas as pl
from jax.experimental.pallas import tpu as pltpu
from jax.experimental.pallas import tpu_sc as plsc

N = 10000
NPAD = 10240
D = 256
E = 160000
KCH = 64
NCH = 168
ET = NCH * KCH
V = 100000
EV = 20
NT = 16
RPT = NPAD // NT
L = 16

f32 = jnp.float32
i32 = jnp.int32

_mesh = plsc.VectorSubcoreMesh(core_axis_name="c", subcore_axis_name="s")
_sc_params = pltpu.CompilerParams(needs_layout_passes=False)


def _bcast_lane(vec, lane):
    return lax.broadcast_in_dim(
        lax.squeeze(lax.slice_in_dim(vec, lane, lane + 1), (0,)), (L,), ())


@functools.partial(
    pl.kernel,
    out_type=jax.ShapeDtypeStruct((2, NPAD, D), f32),
    mesh=_mesh,
    compiler_params=_sc_params,
    scratch_types=[
        pltpu.VMEM((RPT,), i32),
        pltpu.VMEM((160, D), f32),
        pltpu.VMEM((160, D), f32),
        pltpu.SemaphoreType.DMA,
        pltpu.SemaphoreType.DMA,
    ],
)
def _embed_gather(embed_hbm, idx_hbm, out_hbm, idx_v, rows0, rows1, sem0, sem1):
    c = lax.axis_index("c")
    s = lax.axis_index("s")
    base = s * RPT
    pltpu.sync_copy(idx_hbm.at[c, pl.ds(base, RPT)], idx_v)

    bufs = ((rows0, sem0), (rows1, sem1))

    def _issue(i, b):
        rows, sem = bufs[b]
        pltpu.async_copy(embed_hbm.at[idx_v.at[pl.ds(i * 160, 160)]], rows, sem)

    _issue(0, 0)
    _issue(1, 1)
    for i in range(RPT // 160):
        rows, sem = bufs[i % 2]
        pltpu.make_async_copy(
            embed_hbm.at[idx_v.at[pl.ds(i * 160, 160)]], rows, sem).wait()
        pltpu.sync_copy(rows, out_hbm.at[c, pl.ds(base + i * 160, 160)])
        if i + 2 < RPT // 160:
            _issue(i + 2, i % 2)


def _xw_body(x_ref, w_ref, ee_ref, h0_ref, h1_ref, rmax_ref):
    xb = x_ref[0]
    h0_ref[0] = jnp.dot(xb, w_ref[:, :128], preferred_element_type=f32)
    h1_ref[0] = jnp.dot(xb, w_ref[:, 128:], preferred_element_type=f32)
    rm = jnp.max(ee_ref[...], axis=1)
    rmax_ref[...] = jnp.concatenate(
        [rm, jnp.ones((1,), f32), jnp.zeros((127 - EV,), f32)]).reshape(1, 128)


def _dense_xw(x12, W, edge_embed):
    nblk = 8
    rb = NPAD // nblk
    h0, h1, rmax = pl.pallas_call(
        _xw_body,
        grid=(2, nblk),
        in_specs=[
            pl.BlockSpec((1, rb, D), lambda b, i: (b, i, 0)),
            pl.BlockSpec((D, D), lambda b, i: (0, 0)),
            pl.BlockSpec((EV, D), lambda b, i: (0, 0)),
        ],
        out_specs=[
            pl.BlockSpec((1, rb, 128), lambda b, i: (b, i, 0)),
            pl.BlockSpec((1, rb, 128), lambda b, i: (b, i, 0)),
            pl.BlockSpec((1, 128), lambda b, i: (0, 0)),
        ],
        out_shape=[
            jax.ShapeDtypeStruct((2, NPAD, 128), f32),
            jax.ShapeDtypeStruct((2, NPAD, 128), f32),
            jax.ShapeDtypeStruct((1, 128), f32),
        ],
    )(x12, W, edge_embed)
    return h0, h1, rmax


@functools.partial(
    pl.kernel,
    out_type=jax.ShapeDtypeStruct((2, NT, NCH, 1, KCH), f32),
    mesh=_mesh,
    compiler_params=_sc_params,
    scratch_types=[
        pltpu.VMEM((NCH, KCH), i32),
        pltpu.VMEM((NCH, KCH), i32),
        pltpu.VMEM((NCH, KCH), i32),
        pltpu.VMEM((NCH, 1, KCH), f32),
        pltpu.VMEM((NPAD,), f32),
        pltpu.VMEM((80, 128), f32),
        pltpu.VMEM((80, 128), f32),
        pltpu.VMEM((NPAD,), f32),
        pltpu.VMEM((1, 80), i32),
        pltpu.VMEM((128,), f32),
        pltpu.VMEM_SHARED((80, 128), f32),
    ],
)
def _deg_norm(srcs, dsts, eas, rmax_hbm, norm_hbm,
              srcb, dstb, eab, normb, hist, hist2, degb, disb, idx80,
              rmaxb, deg_sh):
    c = lax.axis_index("c")
    s = lax.axis_index("s")

    pltpu.sync_copy(srcs.at[c, s], srcb)
    pltpu.sync_copy(dsts.at[c, s], dstb)
    pltpu.sync_copy(eas.at[c, s], eab)
    pltpu.sync_copy(rmax_hbm, rmaxb)

    for q in range(5):
        idx80[0, pl.ds(q * L, L)] = lax.broadcasted_iota(i32, (L,), 0) + q * L

    def zrow(r, _):
        hist[pl.ds(r * L, L)] = jnp.zeros((L,), f32)
        return ()
    lax.fori_loop(0, NPAD // L, zrow, ())

    def z2row(r, _):
        for q in range(8):
            hist2[r, pl.ds(q * L, L)] = jnp.zeros((L,), f32)
        return ()
    lax.fori_loop(0, 80, z2row, ())

    @pl.when(s == 0)
    def _():
        pltpu.sync_copy(hist2, deg_sh)
    plsc.subcore_barrier()

    def hrow(r, _):
        for q in range(KCH // L):
            dv = dstb[r, pl.ds(q * L, L)]
            av = eab[r, pl.ds(q * L, L)]
            wv = plsc.load_gather(rmaxb, [av])
            plsc.addupdate_scatter(hist, [dv], wv)
        return ()
    lax.fori_loop(0, NCH, hrow, ())

    def crow(r, _):
        for q in range(8):
            hist2[r, pl.ds(q * L, L)] = hist[pl.ds(r * 128 + q * L, L)]
        return ()
    lax.fori_loop(0, 80, crow, ())
    pltpu.sync_copy(hist2, deg_sh.at[idx80.at[0]], add=True)
    plsc.subcore_barrier()

    pltpu.sync_copy(deg_sh, degb)

    def drow(r, _):
        for q in range(8):
            d = degb[r, pl.ds(q * L, L)]
            pos = d > 0.0
            dsafe = jnp.where(pos, d, jnp.ones((L,), f32))
            y = plsc.bitcast(jnp.int32(0x5F3759DF) - (plsc.bitcast(dsafe, i32) >> 1), f32)
            for _ in range(3):
                y = y * (1.5 - 0.5 * dsafe * y * y)
            dis = jnp.where(pos, y, jnp.zeros((L,), f32))
            disb[pl.ds(r * 128 + q * L, L)] = dis
        return ()
    lax.fori_loop(0, 80, drow, ())

    def nrow(r, _):
        for q in range(KCH // L):
            sv = srcb[r, pl.ds(q * L, L)]
            dv = dstb[r, pl.ds(q * L, L)]
            av = eab[r, pl.ds(q * L, L)]
            wv = plsc.load_gather(rmaxb, [av])
            ds_ = plsc.load_gather(disb, [sv])
            dd_ = plsc.load_gather(disb, [dv])
            normb[r, 0, pl.ds(q * L, L)] = ds_ * wv * dd_
        return ()
    lax.fori_loop(0, NCH, nrow, ())
    pltpu.sync_copy(normb, norm_hbm.at[c, s])


NBUF = 4
@functools.partial(
    pl.kernel,
    out_type=jax.ShapeDtypeStruct((2, 2, NPAD, 128), f32),
    mesh=_mesh,
    compiler_params=_sc_params,
    scratch_types=[
        pltpu.VMEM((ET,), i32),
        pltpu.VMEM((1, KCH), i32),
        pltpu.VMEM((1, KCH), i32),
        pltpu.VMEM((1, KCH), i32),
        pltpu.VMEM((1, KCH), i32),
        pltpu.VMEM((KCH,), f32),
        pltpu.VMEM((KCH,), f32),
        pltpu.VMEM((KCH,), f32),
        pltpu.VMEM((KCH,), f32),
        pltpu.VMEM((KCH, 128), f32),
        pltpu.VMEM((KCH, 128), f32),
        pltpu.VMEM((KCH, 128), f32),
        pltpu.VMEM((KCH, 128), f32),
        pltpu.VMEM_SHARED((NPAD, 128), f32),
        pltpu.SemaphoreType.DMA,
        pltpu.SemaphoreType.DMA,
        pltpu.SemaphoreType.DMA,
        pltpu.SemaphoreType.DMA,
    ],
)
def _messages(xw0, xw1, srcs, dsts, norm_hbm, m_hbm,
              srcv, dst0, dst1, dst2, dst3, nrm0, nrm1, nrm2, nrm3,
              rows0, rows1, rows2, rows3, acc_sh, sem0, sem1, sem2, sem3):
    c = lax.axis_index("c")
    s = lax.axis_index("s")

    pltpu.sync_copy(srcs.at[c, s], srcv)
    bufs = ((rows0, nrm0, dst0, sem0), (rows1, nrm1, dst1, sem1),
            (rows2, nrm2, dst2, sem2), (rows3, nrm3, dst3, sem3))

    for h, xwh in ((0, xw0), (1, xw1)):
        def _issue(j, buf):
            rows, nrm, dstb, sem = bufs[buf]
            pltpu.async_copy(norm_hbm.at[c, s, j, 0], nrm, sem)
            pltpu.async_copy(dsts.at[c, s, j], dstb, sem)
            pltpu.async_copy(xwh.at[c].at[srcv.at[pl.ds(j * KCH, KCH)]],
                             rows, sem)

        def _wait(j, buf):
            rows, nrm, dstb, sem = bufs[buf]
            pltpu.make_async_copy(norm_hbm.at[c, s, j, 0], nrm, sem).wait()
            pltpu.make_async_copy(dsts.at[c, s, j], dstb, sem).wait()
            pltpu.make_async_copy(xwh.at[c].at[srcv.at[pl.ds(j * KCH, KCH)]],
                                  rows, sem).wait()

        def zrow(r, _):
            for q in range(8):
                rows0[r, pl.ds(q * L, L)] = jnp.zeros((L,), f32)
            return ()
        lax.fori_loop(0, KCH, zrow, ())
        for i in range(RPT // KCH):
            pltpu.sync_copy(rows0, acc_sh.at[pl.ds(s * RPT + i * KCH, KCH)])
        plsc.subcore_barrier()

        for p in range(NBUF - 1):
            _issue(p, p)

        def quad(jj, _):
            for b in range(NBUF):
                j = NBUF * jj + b
                rows, nrm, dstb, sem = bufs[b]

                @pl.when(j + NBUF - 1 < NCH)
                def _():
                    _issue(j + NBUF - 1, (b + NBUF - 1) % NBUF)
                _wait(j, b)

                def sg(g, _):
                    nv = nrm[pl.ds(g * L, L)]
                    for k16 in range(L):
                        nsp = _bcast_lane(nv, k16)
                        k = g * L + k16
                        for q in range(8):
                            rows[k, pl.ds(q * L, L)] = rows[k, pl.ds(q * L, L)] * nsp
                    return ()
                lax.fori_loop(0, KCH // L, sg, ())
                pltpu.sync_copy(rows, acc_sh.at[dstb.at[0]], add=True)
            return ()
        lax.fori_loop(0, NCH // NBUF, quad, (), unroll=False)
        plsc.subcore_barrier()

        pltpu.sync_copy(acc_sh.at[pl.ds(s * RPT, RPT)],
                        m_hbm.at[c, h, pl.ds(s * RPT, RPT)])
        if h == 0:
            plsc.subcore_barrier()


def _attn_body(m_ref, b2_ref, gw_ref, gb_ref, f1w_ref, f1b_ref, f2w_ref,
               f2b_ref, out_ref):
    iota = lax.broadcasted_iota(i32, (NPAD, 1), 0)
    mask = iota < N
    pooled = []
    for bidx in range(2):
        m0 = jnp.maximum(m_ref[bidx, 0] + b2_ref[0], 0.0)
        m1 = jnp.maximum(m_ref[bidx, 1] + b2_ref[1], 0.0)
        g = (jnp.dot(m0, gw_ref[:128, :], preferred_element_type=f32)
             + jnp.dot(m1, gw_ref[128:, :], preferred_element_type=f32)
             + gb_ref[0, 0])
        g = jax.nn.sigmoid(g)
        gmax = jnp.max(jnp.where(mask, g, -jnp.inf))
        e = jnp.where(mask, jnp.exp(g - gmax), 0.0)
        alpha = e / jnp.sum(e)
        p0 = jnp.sum(alpha * m0, axis=0, keepdims=True)
        p1 = jnp.sum(alpha * m1, axis=0, keepdims=True)
        pooled.append((p0, p1))
    (p10, p11), (p20, p21) = pooled
    h = (jnp.dot(p10, f1w_ref[:128, :], preferred_element_type=f32)
         + jnp.dot(p11, f1w_ref[128:256, :], preferred_element_type=f32)
         + jnp.dot(p20, f1w_ref[256:384, :], preferred_element_type=f32)
         + jnp.dot(p21, f1w_ref[384:, :], preferred_element_type=f32)
         + f1b_ref[...])
    h = jnp.maximum(h, 0.0)
    out_ref[...] = jnp.dot(h, f2w_ref[...], preferred_element_type=f32) + f2b_ref[0, 0]


def _attention_mlp(m, b, gate_W, gate_b, fc1_W, fc1_b, fc2_W, fc2_b):
    return pl.pallas_call(
        _attn_body,
        out_shape=jax.ShapeDtypeStruct((1, 1), f32),
    )(m, b.reshape(2, 128), gate_W, gate_b.reshape(1, 1), fc1_W,
      fc1_b.reshape(1, D), fc2_W, fc2_b.reshape(1, 1))


def _edge_arrays(ei, ea_col):
    self_idx = jnp.arange(NPAD, dtype=i32).reshape(NT, RPT)
    pad_i = jnp.full((NT, 112), N, i32)
    src = jnp.concatenate(
        [ei[0].astype(i32).reshape(NT, E // NT), self_idx, pad_i], axis=1)
    dst = jnp.concatenate(
        [ei[1].astype(i32).reshape(NT, E // NT), self_idx, pad_i], axis=1)
    ea = jnp.concatenate(
        [ea_col.astype(i32).reshape(NT, E // NT),
         jnp.full((NT, RPT), EV, i32), jnp.full((NT, 112), EV, i32)], axis=1)
    return (src.reshape(NT, NCH, KCH), dst.reshape(NT, NCH, KCH),
            ea.reshape(NT, NCH, KCH))


def kernel(x1, x2, edge_index1, edge_index2, edge_attr1, edge_attr2,
           embed, edge_embed, W, b, gate_W, gate_b, fc1_W, fc1_b, fc2_W, fc2_b):
    pad0 = jnp.zeros((NPAD - N,), i32)
    idx = jnp.stack([jnp.concatenate([x1[:, 0].astype(i32), pad0]),
                     jnp.concatenate([x2[:, 0].astype(i32), pad0])])
    s1, d1, a1 = _edge_arrays(edge_index1, edge_attr1[:, 0])
    s2, d2, a2 = _edge_arrays(edge_index2, edge_attr2[:, 0])
    srcs = jnp.stack([s1, s2])
    dsts = jnp.stack([d1, d2])
    eas = jnp.stack([a1, a2])

    x12 = _embed_gather(embed, idx)
    xw0, xw1, rmax = _dense_xw(x12, W, edge_embed)
    norm = _deg_norm(srcs, dsts, eas, rmax.reshape(128))
    m = _messages(xw0, xw1, srcs.reshape(2, NT, ET),
                  dsts.reshape(2, NT, NCH, 1, KCH), norm)
    out = _attention_mlp(m, b, gate_W, gate_b, fc1_W, fc1_b, fc2_W, fc2_b)
    return out.reshape(-1)
